# baseline (device time: 11048 ns/iter reference)
import jax
import jax.numpy as jnp
from jax import lax
from jax.experimental import pallas as pl
from jax.experimental.pallas import tpu as pltpu

N_DEV = 4
NBLK = 8


def kernel(x):
    m_per, n = x.shape
    m_global = N_DEV * m_per
    m_blk = m_per // NBLK

    def body(x_hbm, out_ref, buf_ref, load_sems, comm_ref, send_sems, recv_sems):
        my = lax.axis_index("i")

        def load(b):
            return pltpu.make_async_copy(
                x_hbm.at[pl.ds(b * m_blk, m_blk), :],
                buf_ref.at[b % 2],
                load_sems.at[b % 2],
            )

        load(0).start()

        barrier_sem = pltpu.get_barrier_semaphore()
        for d in range(1, N_DEV):
            pl.semaphore_signal(
                barrier_sem, inc=1,
                device_id=((my + d) % N_DEV,),
                device_id_type=pl.DeviceIdType.MESH,
            )
        pl.semaphore_wait(barrier_sem, N_DEV - 1)

        acc = jnp.zeros((1, n), jnp.float32)
        for b in range(NBLK):
            if b + 1 < NBLK:
                load(b + 1).start()
            load(b).wait()
            acc = acc + jnp.sum(buf_ref[b % 2], axis=0, keepdims=True)

        comm_ref[0, :, :] = acc
        rdmas = []
        for d in range(1, N_DEV):
            rdma = pltpu.make_async_remote_copy(
                src_ref=comm_ref.at[0],
                dst_ref=comm_ref.at[d],
                send_sem=send_sems.at[d - 1],
                recv_sem=recv_sems.at[d - 1],
                device_id=((my + d) % N_DEV,),
                device_id_type=pl.DeviceIdType.MESH,
            )
            rdma.start()
            rdmas.append(rdma)
        for rdma in rdmas:
            rdma.wait()

        total = comm_ref[0, :, :]
        for d in range(1, N_DEV):
            total = total + comm_ref[d, :, :]
        out_ref[:, :] = total * (1.0 / m_global)

    return pl.pallas_call(
        body,
        out_shape=jax.ShapeDtypeStruct((1, n), jnp.float32),
        in_specs=[pl.BlockSpec(memory_space=pltpu.MemorySpace.HBM)],
        out_specs=pl.BlockSpec(memory_space=pltpu.VMEM),
        scratch_shapes=[
            pltpu.VMEM((2, m_blk, n), jnp.float32),
            pltpu.SemaphoreType.DMA((2,)),
            pltpu.VMEM((N_DEV, 1, n), jnp.float32),
            pltpu.SemaphoreType.DMA((N_DEV - 1,)),
            pltpu.SemaphoreType.DMA((N_DEV - 1,)),
        ],
        compiler_params=pltpu.CompilerParams(collective_id=0),
    )(pltpu.with_memory_space_constraint(x, pltpu.MemorySpace.HBM))


# device time: 8791 ns/iter; 1.2567x vs baseline; 1.2567x over previous
import jax
import jax.numpy as jnp
from jax import lax
from jax.experimental import pallas as pl
from jax.experimental.pallas import tpu as pltpu

N_DEV = 4
NBLK = 8


def kernel(x):
    m_per, n = x.shape
    m_global = N_DEV * m_per
    m_blk = m_per // NBLK

    def body(x_hbm, out_ref, buf_ref, load_sems, comm_ref, send_sems, recv_sems):
        my = lax.axis_index("i")

        loads = [
            pltpu.make_async_copy(
                x_hbm.at[pl.ds(b * m_blk, m_blk), :],
                buf_ref.at[b],
                load_sems.at[b],
            )
            for b in range(NBLK)
        ]
        for ld in loads:
            ld.start()

        barrier_sem = pltpu.get_barrier_semaphore()
        for d in range(1, N_DEV):
            pl.semaphore_signal(
                barrier_sem, inc=1,
                device_id=((my + d) % N_DEV,),
                device_id_type=pl.DeviceIdType.MESH,
            )
        pl.semaphore_wait(barrier_sem, N_DEV - 1)

        acc = jnp.zeros((1, n), jnp.float32)
        for b in range(NBLK):
            loads[b].wait()
            acc = acc + jnp.sum(buf_ref[b], axis=0, keepdims=True)

        comm_ref[0, :, :] = acc
        order = [2, 1, 3]
        rdmas = {}
        for d in order:
            rdmas[d] = pltpu.make_async_remote_copy(
                src_ref=comm_ref.at[0],
                dst_ref=comm_ref.at[d],
                send_sem=send_sems.at[d - 1],
                recv_sem=recv_sems.at[d - 1],
                device_id=((my + d) % N_DEV,),
                device_id_type=pl.DeviceIdType.MESH,
            )
            rdmas[d].start()

        total = comm_ref[0, :, :]
        for d in [1, 3, 2]:
            rdmas[d].wait_recv()
            total = total + comm_ref[d, :, :]
        out_ref[:, :] = total * (1.0 / m_global)
        for d in order:
            rdmas[d].wait_send()

    return pl.pallas_call(
        body,
        out_shape=jax.ShapeDtypeStruct((1, n), jnp.float32),
        in_specs=[pl.BlockSpec(memory_space=pltpu.MemorySpace.HBM)],
        out_specs=pl.BlockSpec(memory_space=pltpu.VMEM),
        scratch_shapes=[
            pltpu.VMEM((NBLK, m_blk, n), jnp.float32),
            pltpu.SemaphoreType.DMA((NBLK,)),
            pltpu.VMEM((N_DEV, 1, n), jnp.float32),
            pltpu.SemaphoreType.DMA((N_DEV - 1,)),
            pltpu.SemaphoreType.DMA((N_DEV - 1,)),
        ],
        compiler_params=pltpu.CompilerParams(collective_id=0),
    )(pltpu.with_memory_space_constraint(x, pltpu.MemorySpace.HBM))


# device time: 8761 ns/iter; 1.2610x vs baseline; 1.0034x over previous
import jax
import jax.numpy as jnp
from jax import lax
from jax.experimental import pallas as pl
from jax.experimental.pallas import tpu as pltpu

N_DEV = 4
NBLK = 8


def kernel(x):
    m_per, n = x.shape
    m_global = N_DEV * m_per
    m_blk = m_per // NBLK

    def body(x_hbm, out_ref, buf_ref, load_sems, comm_ref, send_sems, recv_sems):
        my = lax.axis_index("i")

        loads = [
            pltpu.make_async_copy(
                x_hbm.at[pl.ds(b * m_blk, m_blk), :],
                buf_ref.at[b],
                load_sems.at[b],
            )
            for b in range(NBLK)
        ]
        for ld in loads:
            ld.start()

        barrier_sem = pltpu.get_barrier_semaphore()
        for d in range(1, N_DEV):
            pl.semaphore_signal(
                barrier_sem, inc=1,
                device_id=((my + d) % N_DEV,),
                device_id_type=pl.DeviceIdType.MESH,
            )

        acc = jnp.zeros((1, n), jnp.float32)
        for b in range(NBLK):
            loads[b].wait()
            acc = acc + jnp.sum(buf_ref[b], axis=0, keepdims=True)

        pl.semaphore_wait(barrier_sem, N_DEV - 1)

        comm_ref[0, :, :] = acc
        order = [2, 1, 3]
        rdmas = {}
        for d in order:
            rdmas[d] = pltpu.make_async_remote_copy(
                src_ref=comm_ref.at[0],
                dst_ref=comm_ref.at[d],
                send_sem=send_sems.at[d - 1],
                recv_sem=recv_sems.at[d - 1],
                device_id=((my + d) % N_DEV,),
                device_id_type=pl.DeviceIdType.MESH,
            )
            rdmas[d].start()

        total = comm_ref[0, :, :]
        for d in [1, 3, 2]:
            rdmas[d].wait_recv()
            total = total + comm_ref[d, :, :]
        out_ref[:, :] = total * (1.0 / m_global)
        for d in order:
            rdmas[d].wait_send()

    return pl.pallas_call(
        body,
        out_shape=jax.ShapeDtypeStruct((1, n), jnp.float32),
        in_specs=[pl.BlockSpec(memory_space=pltpu.MemorySpace.HBM)],
        out_specs=pl.BlockSpec(memory_space=pltpu.VMEM),
        scratch_shapes=[
            pltpu.VMEM((NBLK, m_blk, n), jnp.float32),
            pltpu.SemaphoreType.DMA((NBLK,)),
            pltpu.VMEM((N_DEV, 1, n), jnp.float32),
            pltpu.SemaphoreType.DMA((N_DEV - 1,)),
            pltpu.SemaphoreType.DMA((N_DEV - 1,)),
        ],
        compiler_params=pltpu.CompilerParams(collective_id=0),
    )(pltpu.with_memory_space_constraint(x, pltpu.MemorySpace.HBM))
